# baseline (speedup 1.0000x reference)
import jax
import jax.numpy as jnp
from jax import lax
from jax.experimental import pallas as pl
from jax.experimental.pallas import tpu as pltpu
from jax.experimental.pallas import tpu_sc as plsc

N = 10000
E = 320000
D_ANN = 512
D_H = 128
L = 8
NSEL = 4096

HALF = N // 2
A_PAD = 5120
TRASH = 5056
CHUNK = 128
G = 7
NG = 6
CPT = G * NG
EPT = CPT * CHUNK
C4 = 16 * EPT
AGG_RPT = A_PAD // 16
MSH_RPT = 320

BM = 400
GRID_M = N // BM
NW = 32


def _sc_scatter_body(m_hbm, edgesA_hbm, edgesB_hbm, zeros_hbm, out_hbm,
                     idxA_v, idxB_v, rowsA, rowsB, m_sh, agg_sh,
                     semiA, semiB, semB):
    c = lax.axis_index("c")
    s = lax.axis_index("s")

    pltpu.sync_copy(edgesA_hbm.at[c, s, 0], idxA_v.at[0])
    pltpu.sync_copy(edgesB_hbm.at[c, s, 0], idxB_v.at[0])

    r0 = s * MSH_RPT

    @pl.when(s < 15)
    def _():
        pltpu.sync_copy(m_hbm.at[pl.ds(c * HALF + r0, MSH_RPT)],
                        m_sh.at[pl.ds(r0, MSH_RPT)])

    @pl.when(s == 15)
    def _():
        pltpu.sync_copy(m_hbm.at[pl.ds(c * HALF + r0, HALF - 15 * MSH_RPT)],
                        m_sh.at[pl.ds(r0, HALF - 15 * MSH_RPT)])

    pltpu.sync_copy(zeros_hbm, agg_sh.at[pl.ds(s * AGG_RPT, AGG_RPT)])
    plsc.subcore_barrier()

    def group(g, carry):
        gb = lax.rem(g, 2)

        @pl.when(g + 1 < NG)
        def _():
            pltpu.async_copy(edgesA_hbm.at[c, s, g + 1], idxA_v.at[1 - gb],
                             semiA)
            pltpu.async_copy(edgesB_hbm.at[c, s, g + 1], idxB_v.at[1 - gb],
                             semiB)

        for u in range(G):
            pltpu.async_copy(m_hbm.at[idxB_v.at[gb, 0, u]], rowsB, semB)
            pltpu.sync_copy(m_sh.at[idxA_v.at[gb, 0, u]], rowsA)
            pltpu.sync_copy(rowsA, agg_sh.at[idxA_v.at[gb, 1, u]],
                            add=True)
            pltpu.make_async_copy(m_hbm.at[idxB_v.at[gb, 0, u]], rowsB,
                                  semB).wait()
            pltpu.sync_copy(rowsB, agg_sh.at[idxB_v.at[gb, 1, u]],
                            add=True)

        @pl.when(g + 1 < NG)
        def _():
            pltpu.make_async_copy(edgesA_hbm.at[c, s, g + 1],
                                  idxA_v.at[1 - gb], semiA).wait()
            pltpu.make_async_copy(edgesB_hbm.at[c, s, g + 1],
                                  idxB_v.at[1 - gb], semiB).wait()
        return carry

    lax.fori_loop(0, NG, group, 0)
    plsc.subcore_barrier()

    @pl.when(s < 15)
    def _():
        pltpu.sync_copy(agg_sh.at[pl.ds(s * AGG_RPT, AGG_RPT)],
                        out_hbm.at[pl.ds(c * HALF + s * AGG_RPT, AGG_RPT)])

    @pl.when(s == 15)
    def _():
        pltpu.sync_copy(agg_sh.at[pl.ds(s * AGG_RPT, HALF - 15 * AGG_RPT)],
                        out_hbm.at[pl.ds(c * HALF + s * AGG_RPT,
                                         HALF - 15 * AGG_RPT)])


def _make_sc_scatter():
    return pl.kernel(
        _sc_scatter_body,
        out_type=jax.ShapeDtypeStruct((N, D_H), jnp.float32),
        mesh=plsc.VectorSubcoreMesh(core_axis_name="c", subcore_axis_name="s"),
        scratch_types=[
            pltpu.VMEM((2, 2, G, CHUNK), jnp.int32),
            pltpu.VMEM((2, 2, G, CHUNK), jnp.int32),
            pltpu.VMEM((CHUNK, D_H), jnp.float32),
            pltpu.VMEM((CHUNK, D_H), jnp.float32),
            pltpu.VMEM_SHARED((HALF, D_H), jnp.float32),
            pltpu.VMEM_SHARED((A_PAD, D_H), jnp.float32),
            pltpu.SemaphoreType.DMA,
            pltpu.SemaphoreType.DMA,
            pltpu.SemaphoreType.DMA,
        ],
    )


def _sc_gather_body(h_hbm, idx_hbm, sel_hbm, idx_v, rows_v, sem):
    c = lax.axis_index("c")
    s = lax.axis_index("s")
    base = (c * 16 + s) * (NSEL // NW)
    pltpu.sync_copy(idx_hbm.at[pl.ds(base, NSEL // NW)], idx_v)
    pltpu.async_copy(h_hbm.at[idx_v], rows_v, sem).wait()
    pltpu.sync_copy(rows_v, sel_hbm.at[pl.ds(base, NSEL // NW)])


def _make_sc_gather():
    return pl.kernel(
        _sc_gather_body,
        out_type=jax.ShapeDtypeStruct((NSEL, D_H), jnp.float32),
        mesh=plsc.VectorSubcoreMesh(core_axis_name="c", subcore_axis_name="s"),
        scratch_types=[
            pltpu.VMEM((NSEL // NW,), jnp.int32),
            pltpu.VMEM((NSEL // NW, D_H), jnp.float32),
            pltpu.SemaphoreType.DMA,
        ],
    )


def _k0_body(x_ref, wred_ref, bred_ref, w0_ref, whhT_ref, bhh_ref,
             h_ref, m_ref, gh_ref):
    h = jnp.dot(x_ref[...], wred_ref[...],
                preferred_element_type=jnp.float32) + bred_ref[...]
    h_ref[...] = h
    m_ref[...] = jnp.dot(h, w0_ref[...], preferred_element_type=jnp.float32)
    gh_ref[...] = jnp.dot(h, whhT_ref[...],
                          preferred_element_type=jnp.float32) + bhh_ref[...]


def _gru_body(agg_ref, gh_ref, h_ref, wihT_ref, bih_ref,
              wnext_ref, whhT_ref, bhh_ref,
              hN_ref, mN_ref, ghN_ref):
    agg = agg_ref[...]
    gi = jnp.dot(agg, wihT_ref[...],
                 preferred_element_type=jnp.float32) + bih_ref[...]
    gh = gh_ref[...]
    h = h_ref[...]
    r = jax.nn.sigmoid(gi[:, :D_H] + gh[:, :D_H])
    z = jax.nn.sigmoid(gi[:, D_H:2 * D_H] + gh[:, D_H:2 * D_H])
    n = jnp.tanh(gi[:, 2 * D_H:] + r * gh[:, 2 * D_H:])
    hn = (1.0 - z) * n + z * h
    hN_ref[...] = hn
    mN_ref[...] = jnp.dot(hn, wnext_ref[...], preferred_element_type=jnp.float32)
    ghN_ref[...] = jnp.dot(hn, whhT_ref[...],
                           preferred_element_type=jnp.float32) + bhh_ref[...]


def _gru_last_body(agg_ref, gh_ref, h_ref, wihT_ref, bih_ref,
                   hN_ref):
    agg = agg_ref[...]
    gi = jnp.dot(agg, wihT_ref[...],
                 preferred_element_type=jnp.float32) + bih_ref[...]
    gh = gh_ref[...]
    h = h_ref[...]
    r = jax.nn.sigmoid(gi[:, :D_H] + gh[:, :D_H])
    z = jax.nn.sigmoid(gi[:, D_H:2 * D_H] + gh[:, D_H:2 * D_H])
    n = jnp.tanh(gi[:, 2 * D_H:] + r * gh[:, 2 * D_H:])
    hN_ref[...] = (1.0 - z) * n + z * h


def _final_body(sel_ref, wlin_ref, blin_ref, out_ref):
    s = jax.nn.sigmoid(sel_ref[...])
    out_ref[...] = jax.nn.sigmoid(
        jnp.dot(s, wlin_ref[...], preferred_element_type=jnp.float32)
        + blin_ref[...])


def _row_spec(bm, d):
    return pl.BlockSpec((bm, d), lambda i: (i, 0))


def _full_spec(shape):
    return pl.BlockSpec(shape, lambda i: tuple(0 for _ in shape))


def _partition_edges(src, dst):
    i32 = jnp.int32
    dhalf = (dst >= HALF).astype(i32)
    shalf = (src >= HALF).astype(i32)
    sec = dhalf * 2 + shalf
    pos_in = jnp.zeros_like(src)
    base = jnp.zeros_like(src)
    for k in range(4):
        mk = sec == k
        pos_in = jnp.where(mk, jnp.cumsum(mk.astype(i32)) - 1, pos_in)
        base = jnp.where(mk, k * C4, base)
    pos = jnp.where(pos_in < C4, base + pos_in, 4 * C4)
    src_l = jnp.where(shalf == 1, src - HALF, src)
    dst_l = jnp.where(dhalf == 1, dst - HALF, dst)
    Sl = jnp.zeros((4 * C4,), i32).at[pos].set(src_l, mode="drop")
    Sg = jnp.zeros((4 * C4,), i32).at[pos].set(src, mode="drop")
    Dd = jnp.full((4 * C4,), TRASH, i32).at[pos].set(dst_l, mode="drop")

    def sect(arr, k):
        return arr[k * C4:(k + 1) * C4].reshape(16, NG, G, CHUNK)

    srcA = jnp.stack([sect(Sl, 0), sect(Sl, 3)])
    dstA = jnp.stack([sect(Dd, 0), sect(Dd, 3)])
    srcB = jnp.stack([sect(Sg, 1), sect(Sg, 2)])
    dstB = jnp.stack([sect(Dd, 1), sect(Dd, 2)])
    edgesA = jnp.stack([srcA, dstA], axis=3)
    edgesB = jnp.stack([srcB, dstB], axis=3)
    return edgesA, edgesB


def kernel(x, edge_index, idx, W_reduce, b_reduce, weight, W_ih, W_hh,
           b_ih, b_hh, W_lin, b_lin):
    f32 = jnp.float32
    edgesA, edgesB = _partition_edges(edge_index[0], edge_index[1])
    zeros_stage = jnp.zeros((AGG_RPT, D_H), f32)

    W_ihT = W_ih.T
    W_hhT = W_hh.T
    bih_r = b_ih.reshape(1, 3 * D_H)
    bhh_r = b_hh.reshape(1, 3 * D_H)
    bred_r = b_reduce.reshape(1, D_H)
    wlin_p = jnp.zeros((D_H, D_H), f32).at[:, :1].set(W_lin)
    blin_p = jnp.zeros((1, D_H), f32).at[0, 0].set(b_lin[0])

    k0 = pl.pallas_call(
        _k0_body,
        grid=(GRID_M,),
        in_specs=[
            _row_spec(BM, D_ANN),
            _full_spec((D_ANN, D_H)),
            _full_spec((1, D_H)),
            _full_spec((D_H, D_H)),
            _full_spec((D_H, 3 * D_H)),
            _full_spec((1, 3 * D_H)),
        ],
        out_specs=[
            _row_spec(BM, D_H),
            _row_spec(BM, D_H),
            _row_spec(BM, 3 * D_H),
        ],
        out_shape=[
            jax.ShapeDtypeStruct((N, D_H), f32),
            jax.ShapeDtypeStruct((N, D_H), f32),
            jax.ShapeDtypeStruct((N, 3 * D_H), f32),
        ],
    )
    h, m, gh = k0(x, W_reduce, bred_r, weight[0], W_hhT, bhh_r)

    sc_scatter = _make_sc_scatter()
    sc_gather = _make_sc_gather()

    gru_mid = pl.pallas_call(
        _gru_body,
        grid=(GRID_M,),
        in_specs=[
            _row_spec(BM, D_H),
            _row_spec(BM, 3 * D_H),
            _row_spec(BM, D_H),
            _full_spec((D_H, 3 * D_H)),
            _full_spec((1, 3 * D_H)),
            _full_spec((D_H, D_H)),
            _full_spec((D_H, 3 * D_H)),
            _full_spec((1, 3 * D_H)),
        ],
        out_specs=[
            _row_spec(BM, D_H),
            _row_spec(BM, D_H),
            _row_spec(BM, 3 * D_H),
        ],
        out_shape=[
            jax.ShapeDtypeStruct((N, D_H), f32),
            jax.ShapeDtypeStruct((N, D_H), f32),
            jax.ShapeDtypeStruct((N, 3 * D_H), f32),
        ],
    )
    gru_last = pl.pallas_call(
        _gru_last_body,
        grid=(GRID_M,),
        in_specs=[
            _row_spec(BM, D_H),
            _row_spec(BM, 3 * D_H),
            _row_spec(BM, D_H),
            _full_spec((D_H, 3 * D_H)),
            _full_spec((1, 3 * D_H)),
        ],
        out_specs=_row_spec(BM, D_H),
        out_shape=jax.ShapeDtypeStruct((N, D_H), f32),
    )

    for i in range(L):
        agg = sc_scatter(m, edgesA, edgesB, zeros_stage)
        if i < L - 1:
            h, m, gh = gru_mid(agg, gh, h, W_ihT, bih_r,
                               weight[i + 1], W_hhT, bhh_r)
        else:
            h = gru_last(agg, gh, h, W_ihT, bih_r)

    sel = sc_gather(h, idx)

    final = pl.pallas_call(
        _final_body,
        grid=(NSEL // 512,),
        in_specs=[
            _row_spec(512, D_H),
            _full_spec((D_H, D_H)),
            _full_spec((1, D_H)),
        ],
        out_specs=_row_spec(512, D_H),
        out_shape=jax.ShapeDtypeStruct((NSEL, D_H), f32),
    )
    out_full = final(sel, wlin_p, blin_p)
    return out_full[:, :1]


# speedup vs baseline: 2.1256x; 2.1256x over previous
import functools

import jax
import jax.numpy as jnp
from jax import lax
from jax.experimental import pallas as pl
from jax.experimental.pallas import tpu as pltpu
from jax.experimental.pallas import tpu_sc as plsc

N = 10000
E = 320000
D_ANN = 512
D_H = 128
L = 8
NSEL = 4096

N_PAD = 10240
E_PAD = 327680
NW = 32
EDGES_PER_W = E_PAD // NW
CHUNK = 128
CHUNKS_PER_W = EDGES_PER_W // CHUNK
ROWS_PER_TILE = N_PAD // 16

BM = 400
GRID_M = N // BM


G = 8
NG = CHUNKS_PER_W // G


def _sc_scatter_body(m_hbm, edges_hbm, zeros_hbm, out_hbm,
                     edges_v, rows0, rows1, agg_sh,
                     semi, semg0, semg1):
    c = lax.axis_index("c")
    s = lax.axis_index("s")
    wid = c * 16 + s
    rows = (rows0, rows1)
    semg = (semg0, semg1)

    pltpu.sync_copy(edges_hbm.at[wid, 0], edges_v.at[0])
    pltpu.async_copy(m_hbm.at[edges_v.at[0, 0, 0]], rows0, semg0)

    row0 = s * ROWS_PER_TILE
    pltpu.sync_copy(zeros_hbm, rows1)
    for k in range(ROWS_PER_TILE // CHUNK):
        pltpu.sync_copy(rows1, agg_sh.at[pl.ds(row0 + k * CHUNK, CHUNK)])
    plsc.subcore_barrier()

    def group(g, carry):
        gb = lax.rem(g, 2)

        @pl.when(g + 1 < NG)
        def _():
            pltpu.async_copy(edges_hbm.at[wid, g + 1], edges_v.at[1 - gb],
                             semi)

        for k in range(G):
            b = k % 2
            pltpu.make_async_copy(m_hbm.at[edges_v.at[gb, 0, k]], rows[b],
                                  semg[b]).wait()
            if k + 1 < G:
                pltpu.async_copy(m_hbm.at[edges_v.at[gb, 0, k + 1]],
                                 rows[1 - b], semg[1 - b])
            else:
                @pl.when(g + 1 < NG)
                def _():
                    pltpu.make_async_copy(edges_hbm.at[wid, g + 1],
                                          edges_v.at[1 - gb], semi).wait()
                    pltpu.async_copy(m_hbm.at[edges_v.at[1 - gb, 0, 0]],
                                     rows[1 - b], semg[1 - b])
            pltpu.sync_copy(rows[b], agg_sh.at[edges_v.at[gb, 1, k]],
                            add=True)
        return carry

    lax.fori_loop(0, NG, group, 0)
    plsc.subcore_barrier()

    for k in range(ROWS_PER_TILE // CHUNK):
        r = row0 + k * CHUNK
        pltpu.sync_copy(agg_sh.at[pl.ds(r, CHUNK)], rows0)

        @pl.when(r + CHUNK <= N)
        def _():
            pltpu.sync_copy(rows0, out_hbm.at[pl.ds(c * N + r, CHUNK)])

        @pl.when(jnp.logical_and(r < N, r + CHUNK > N))
        def _():
            pltpu.sync_copy(rows0.at[pl.ds(0, N % CHUNK)],
                            out_hbm.at[pl.ds(c * N + r, N % CHUNK)])


def _make_sc_scatter():
    return pl.kernel(
        _sc_scatter_body,
        out_type=jax.ShapeDtypeStruct((2 * N, D_H), jnp.float32),
        mesh=plsc.VectorSubcoreMesh(core_axis_name="c", subcore_axis_name="s"),
        scratch_types=[
            pltpu.VMEM((2, 2, G, CHUNK), jnp.int32),
            pltpu.VMEM((CHUNK, D_H), jnp.float32),
            pltpu.VMEM((CHUNK, D_H), jnp.float32),
            pltpu.VMEM_SHARED((N_PAD, D_H), jnp.float32),
            pltpu.SemaphoreType.DMA,
            pltpu.SemaphoreType.DMA,
            pltpu.SemaphoreType.DMA,
        ],
    )


def _sc_gather_body(h_hbm, idx_hbm, sel_hbm, idx_v, rows_v, sem):
    c = lax.axis_index("c")
    s = lax.axis_index("s")
    base = (c * 16 + s) * (NSEL // NW)
    pltpu.sync_copy(idx_hbm.at[pl.ds(base, NSEL // NW)], idx_v)
    pltpu.async_copy(h_hbm.at[idx_v], rows_v, sem).wait()
    pltpu.sync_copy(rows_v, sel_hbm.at[pl.ds(base, NSEL // NW)])


def _make_sc_gather():
    return pl.kernel(
        _sc_gather_body,
        out_type=jax.ShapeDtypeStruct((NSEL, D_H), jnp.float32),
        mesh=plsc.VectorSubcoreMesh(core_axis_name="c", subcore_axis_name="s"),
        scratch_types=[
            pltpu.VMEM((NSEL // NW,), jnp.int32),
            pltpu.VMEM((NSEL // NW, D_H), jnp.float32),
            pltpu.SemaphoreType.DMA,
        ],
    )


def _k0_body(x_ref, wred_ref, bred_ref, w0_ref, whhT_ref, bhh_ref,
             h_ref, m_ref, gh_ref):
    h = jnp.dot(x_ref[...], wred_ref[...],
                preferred_element_type=jnp.float32) + bred_ref[...]
    h_ref[...] = h
    m_ref[...] = jnp.dot(h, w0_ref[...], preferred_element_type=jnp.float32)
    gh_ref[...] = jnp.dot(h, whhT_ref[...],
                          preferred_element_type=jnp.float32) + bhh_ref[...]


def _gru_body(p0_ref, p1_ref, gh_ref, h_ref, wihT_ref, bih_ref,
              wnext_ref, whhT_ref, bhh_ref,
              hN_ref, mN_ref, ghN_ref):
    agg = p0_ref[...] + p1_ref[...]
    gi = jnp.dot(agg, wihT_ref[...],
                 preferred_element_type=jnp.float32) + bih_ref[...]
    gh = gh_ref[...]
    h = h_ref[...]
    r = jax.nn.sigmoid(gi[:, :D_H] + gh[:, :D_H])
    z = jax.nn.sigmoid(gi[:, D_H:2 * D_H] + gh[:, D_H:2 * D_H])
    n = jnp.tanh(gi[:, 2 * D_H:] + r * gh[:, 2 * D_H:])
    hn = (1.0 - z) * n + z * h
    hN_ref[...] = hn
    mN_ref[...] = jnp.dot(hn, wnext_ref[...], preferred_element_type=jnp.float32)
    ghN_ref[...] = jnp.dot(hn, whhT_ref[...],
                           preferred_element_type=jnp.float32) + bhh_ref[...]


def _gru_last_body(p0_ref, p1_ref, gh_ref, h_ref, wihT_ref, bih_ref,
                   hN_ref):
    agg = p0_ref[...] + p1_ref[...]
    gi = jnp.dot(agg, wihT_ref[...],
                 preferred_element_type=jnp.float32) + bih_ref[...]
    gh = gh_ref[...]
    h = h_ref[...]
    r = jax.nn.sigmoid(gi[:, :D_H] + gh[:, :D_H])
    z = jax.nn.sigmoid(gi[:, D_H:2 * D_H] + gh[:, D_H:2 * D_H])
    n = jnp.tanh(gi[:, 2 * D_H:] + r * gh[:, 2 * D_H:])
    hN_ref[...] = (1.0 - z) * n + z * h


def _final_body(sel_ref, wlin_ref, blin_ref, out_ref):
    s = jax.nn.sigmoid(sel_ref[...])
    out_ref[...] = jax.nn.sigmoid(
        jnp.dot(s, wlin_ref[...], preferred_element_type=jnp.float32)
        + blin_ref[...])


def _row_spec(bm, d):
    return pl.BlockSpec((bm, d), lambda i: (i, 0))


def _full_spec(shape):
    return pl.BlockSpec(shape, lambda i: tuple(0 for _ in shape))


def kernel(x, edge_index, idx, W_reduce, b_reduce, weight, W_ih, W_hh,
           b_ih, b_hh, W_lin, b_lin):
    f32 = jnp.float32
    src = edge_index[0]
    dst = edge_index[1]
    pad = E_PAD - E
    src_p = jnp.concatenate([src, jnp.zeros((pad,), jnp.int32)])
    dst_p = jnp.concatenate([dst, jnp.full((pad,), N, jnp.int32)])
    edges_p = jnp.stack(
        [src_p.reshape(NW, NG, G, CHUNK), dst_p.reshape(NW, NG, G, CHUNK)],
        axis=2)
    zeros_stage = jnp.zeros((CHUNK, D_H), f32)

    W_ihT = W_ih.T
    W_hhT = W_hh.T
    bih_r = b_ih.reshape(1, 3 * D_H)
    bhh_r = b_hh.reshape(1, 3 * D_H)
    bred_r = b_reduce.reshape(1, D_H)
    wlin_p = jnp.zeros((D_H, D_H), f32).at[:, :1].set(W_lin)
    blin_p = jnp.zeros((1, D_H), f32).at[0, 0].set(b_lin[0])

    k0 = pl.pallas_call(
        _k0_body,
        grid=(GRID_M,),
        in_specs=[
            _row_spec(BM, D_ANN),
            _full_spec((D_ANN, D_H)),
            _full_spec((1, D_H)),
            _full_spec((D_H, D_H)),
            _full_spec((D_H, 3 * D_H)),
            _full_spec((1, 3 * D_H)),
        ],
        out_specs=[
            _row_spec(BM, D_H),
            _row_spec(BM, D_H),
            _row_spec(BM, 3 * D_H),
        ],
        out_shape=[
            jax.ShapeDtypeStruct((N, D_H), f32),
            jax.ShapeDtypeStruct((N, D_H), f32),
            jax.ShapeDtypeStruct((N, 3 * D_H), f32),
        ],
    )
    h, m, gh = k0(x, W_reduce, bred_r, weight[0], W_hhT, bhh_r)

    sc_scatter = _make_sc_scatter()
    sc_gather = _make_sc_gather()

    gru_mid = pl.pallas_call(
        _gru_body,
        grid=(GRID_M,),
        in_specs=[
            pl.BlockSpec((BM, D_H), lambda i: (i, 0)),
            pl.BlockSpec((BM, D_H), lambda i: (i + GRID_M, 0)),
            _row_spec(BM, 3 * D_H),
            _row_spec(BM, D_H),
            _full_spec((D_H, 3 * D_H)),
            _full_spec((1, 3 * D_H)),
            _full_spec((D_H, D_H)),
            _full_spec((D_H, 3 * D_H)),
            _full_spec((1, 3 * D_H)),
        ],
        out_specs=[
            _row_spec(BM, D_H),
            _row_spec(BM, D_H),
            _row_spec(BM, 3 * D_H),
        ],
        out_shape=[
            jax.ShapeDtypeStruct((N, D_H), f32),
            jax.ShapeDtypeStruct((N, D_H), f32),
            jax.ShapeDtypeStruct((N, 3 * D_H), f32),
        ],
    )
    gru_last = pl.pallas_call(
        _gru_last_body,
        grid=(GRID_M,),
        in_specs=[
            pl.BlockSpec((BM, D_H), lambda i: (i, 0)),
            pl.BlockSpec((BM, D_H), lambda i: (i + GRID_M, 0)),
            _row_spec(BM, 3 * D_H),
            _row_spec(BM, D_H),
            _full_spec((D_H, 3 * D_H)),
            _full_spec((1, 3 * D_H)),
        ],
        out_specs=_row_spec(BM, D_H),
        out_shape=jax.ShapeDtypeStruct((N, D_H), f32),
    )

    for i in range(L):
        partials = sc_scatter(m, edges_p, zeros_stage)
        if i < L - 1:
            h, m, gh = gru_mid(partials, partials, gh, h, W_ihT, bih_r,
                               weight[i + 1], W_hhT, bhh_r)
        else:
            h = gru_last(partials, partials, gh, h, W_ihT, bih_r)

    sel = sc_gather(h, idx)

    final = pl.pallas_call(
        _final_body,
        grid=(NSEL // 512,),
        in_specs=[
            _row_spec(512, D_H),
            _full_spec((D_H, D_H)),
            _full_spec((1, D_H)),
        ],
        out_specs=_row_spec(512, D_H),
        out_shape=jax.ShapeDtypeStruct((NSEL, D_H), f32),
    )
    out_full = final(sel, wlin_p, blin_p)
    return out_full[:, :1]
